# Initial kernel scaffold; baseline (speedup 1.0000x reference)
import jax
import jax.numpy as jnp
from jax.experimental import pallas as pl


def kernel(sample, entity_embedding, relation_embedding):
    raise NotImplementedError("write your pallas kernel here")


# speedup vs baseline: 1.6392x; 1.6392x over previous
import functools

import jax
import jax.numpy as jnp
from jax import lax
from jax.experimental import pallas as pl
from jax.experimental.pallas import tpu as pltpu
from jax.experimental.pallas import tpu_sc as plsc

_GAMMA = 12.0
_B = 16384
_D = 128
_L = 16
_NC, _NS = 2, 16
_NW = _NC * _NS
_BPW = _B // _NW
_CHUNK = 128
_NCHUNK = _BPW // _CHUNK
_DV = _D // _L

_mesh = plsc.VectorSubcoreMesh(core_axis_name="c", subcore_axis_name="s")


@functools.partial(
    pl.kernel,
    out_type=jax.ShapeDtypeStruct((_B,), jnp.float32),
    mesh=_mesh,
    scratch_types=[
        pltpu.VMEM((_BPW,), jnp.int32),
        pltpu.VMEM((_BPW,), jnp.int32),
        pltpu.VMEM((_BPW,), jnp.int32),
        pltpu.VMEM((2, _CHUNK, _D), jnp.float32),
        pltpu.VMEM((2, _CHUNK, _D), jnp.float32),
        pltpu.VMEM((2, _CHUNK, _D), jnp.float32),
        pltpu.VMEM((_BPW,), jnp.float32),
        pltpu.SemaphoreType.DMA,
        pltpu.SemaphoreType.DMA,
    ],
)
def _transe_sc(hi_hbm, ri_hbm, ti_hbm, ent_hbm, rel_hbm, out_hbm,
               hi_v, ri_v, ti_v, h_v, r_v, t_v, out_v, sem0, sem1):
    wid = lax.axis_index("s") * _NC + lax.axis_index("c")
    base = wid * _BPW

    pltpu.sync_copy(hi_hbm.at[pl.ds(base, _BPW)], hi_v)
    pltpu.sync_copy(ri_hbm.at[pl.ds(base, _BPW)], ri_v)
    pltpu.sync_copy(ti_hbm.at[pl.ds(base, _BPW)], ti_v)

    sems = (sem0, sem1)

    def start_gathers(c, slot):
        off = c * _CHUNK
        sem = sems[slot]
        d0 = pltpu.async_copy(ent_hbm.at[hi_v.at[pl.ds(off, _CHUNK)]],
                              h_v.at[slot], sem)
        d1 = pltpu.async_copy(rel_hbm.at[ri_v.at[pl.ds(off, _CHUNK)]],
                              r_v.at[slot], sem)
        d2 = pltpu.async_copy(ent_hbm.at[ti_v.at[pl.ds(off, _CHUNK)]],
                              t_v.at[slot], sem)
        return (d0, d1, d2)

    lanes = lax.iota(jnp.int32, _L)

    def compute_chunk(c, slot):
        hs, rs, ts = h_v.at[slot], r_v.at[slot], t_v.at[slot]
        out_off = c * _CHUNK

        def body(g, _):
            i0 = g * _L
            score = jnp.zeros((_L,), jnp.float32)
            for k in range(_L):
                acc = jnp.zeros((_L,), jnp.float32)
                for j in range(_DV):
                    dsl = pl.ds(j * _L, _L)
                    acc = acc + jnp.abs(
                        hs[i0 + k, dsl] + rs[i0 + k, dsl] - ts[i0 + k, dsl])
                s = acc[0]
                for m in range(1, _L):
                    s = s + acc[m]
                score = score + jnp.where(lanes == k, _GAMMA - s, 0.0)
            out_v[pl.ds(out_off + i0, _L)] = score
            return 0

        lax.fori_loop(0, _CHUNK // _L, body, 0)

    pending = start_gathers(0, 0)
    for c in range(_NCHUNK):
        for d in pending:
            d.wait()
        if c + 1 < _NCHUNK:
            pending = start_gathers(c + 1, (c + 1) % 2)
        compute_chunk(c, c % 2)

    pltpu.sync_copy(out_v, out_hbm.at[pl.ds(base, _BPW)])


def kernel(sample, entity_embedding, relation_embedding):
    hi = sample[:, 0]
    ri = sample[:, 1]
    ti = sample[:, 2]
    out = _transe_sc(hi, ri, ti, entity_embedding, relation_embedding)
    return out[:, None]
